# baseline (speedup 1.0000x reference)
import functools

import jax
import jax.numpy as jnp
from jax import lax
from jax.experimental import pallas as pl
from jax.experimental.pallas import tpu as pltpu
from jax.experimental.pallas import tpu_sc as plsc

B = 16384
D = 32
NC = 2
NS = 16
NW = NC * NS
BPW = B // NW
CHUNK = 128
NCHUNK = BPW // CHUNK


def _sc_body(uidx_hbm, iidx_hbm, uln_hbm, iln_hbm, ubf_hbm, ibf_hbm,
             partials_hbm, bias_hbm,
             uidx_v, iidx_v, ugrp_v, igrp_v,
             ubv_v, ibv_v, acc_v, outb_v,
             sem_u, sem_i, sem_ub, sem_ib):
    wid = lax.axis_index("s") * NC + lax.axis_index("c")
    base = wid * BPW
    row0 = wid * NCHUNK

    pltpu.sync_copy(uidx_hbm.at[pl.ds(row0, NCHUNK)], uidx_v)
    pltpu.sync_copy(iidx_hbm.at[pl.ds(row0, NCHUNK)], iidx_v)

    bias_copies = []
    for j in range(NCHUNK):
        bias_copies.append(pltpu.async_copy(
            ubf_hbm.at[uidx_v.at[j]], ubv_v.at[pl.ds(j * CHUNK, CHUNK)],
            sem_ub))
        bias_copies.append(pltpu.async_copy(
            ibf_hbm.at[iidx_v.at[j]], ibv_v.at[pl.ds(j * CHUNK, CHUNK)],
            sem_ib))

    acc = jnp.zeros((16,), jnp.float32)

    for j in range(NCHUNK):
        cu = pltpu.async_copy(uln_hbm.at[uidx_v.at[j]], ugrp_v, sem_u)
        ci = pltpu.async_copy(iln_hbm.at[iidx_v.at[j]], igrp_v, sem_i)
        cu.wait()
        ci.wait()

        def dot_body(n, a):
            u0 = ugrp_v[n, pl.ds(0, 16)]
            v0 = igrp_v[n, pl.ds(0, 16)]
            u1 = ugrp_v[n, pl.ds(16, 16)]
            v1 = igrp_v[n, pl.ds(16, 16)]
            return a + u0 * v0 + u1 * v1

        acc = lax.fori_loop(0, CHUNK, dot_body, acc, unroll=4)

    acc_v[...] = acc
    pltpu.sync_copy(acc_v, partials_hbm.at[wid])

    for c in bias_copies:
        c.wait()
    for j in range(BPW // 16):
        outb_v[pl.ds(j * 16, 16)] = (
            ubv_v[pl.ds(j * 16, 16)] + ibv_v[pl.ds(j * 16, 16)])
    pltpu.sync_copy(outb_v, bias_hbm.at[pl.ds(base, BPW)])


@functools.cache
def _make_sc_call():
    return pl.kernel(
        _sc_body,
        out_type=[
            jax.ShapeDtypeStruct((NW, 16), jnp.float32),
            jax.ShapeDtypeStruct((B,), jnp.float32),
        ],
        mesh=plsc.VectorSubcoreMesh(
            core_axis_name="c", subcore_axis_name="s"),
        compiler_params=pltpu.CompilerParams(
            use_tc_tiling_on_sc=True, needs_layout_passes=False),
        scratch_types=[
            pltpu.VMEM((NCHUNK, CHUNK), jnp.int32),
            pltpu.VMEM((NCHUNK, CHUNK), jnp.int32),
            pltpu.VMEM((CHUNK, 128), jnp.float32),
            pltpu.VMEM((CHUNK, 128), jnp.float32),
            pltpu.VMEM((BPW,), jnp.float32),
            pltpu.VMEM((BPW,), jnp.float32),
            pltpu.VMEM((16,), jnp.float32),
            pltpu.VMEM((BPW,), jnp.float32),
            pltpu.SemaphoreType.DMA,
            pltpu.SemaphoreType.DMA,
            pltpu.SemaphoreType.DMA,
            pltpu.SemaphoreType.DMA,
        ],
    )


PAD_BLK = 8000


def _pad_body(u_in, i_in, u_out, i_out):
    u_out[:, 0:D] = u_in[...]
    i_out[:, 0:D] = i_in[...]


_pad_call = pl.pallas_call(
    _pad_body,
    grid=(1000000 // PAD_BLK,),
    in_specs=[
        pl.BlockSpec((PAD_BLK, D), lambda i: (i, 0)),
        pl.BlockSpec((PAD_BLK, D), lambda i: (i, 0)),
    ],
    out_specs=[
        pl.BlockSpec((PAD_BLK, 128), lambda i: (i, 0)),
        pl.BlockSpec((PAD_BLK, 128), lambda i: (i, 0)),
    ],
    out_shape=[
        jax.ShapeDtypeStruct((1000000, 128), jnp.float32),
        jax.ShapeDtypeStruct((1000000, 128), jnp.float32),
    ],
)


def _tc_body(bias_ref, partials_ref, out_ref):
    total = jnp.sum(partials_ref[...])
    out_ref[...] = bias_ref[...] + total


_tc_call = pl.pallas_call(
    _tc_body,
    out_shape=jax.ShapeDtypeStruct((128, 128), jnp.float32),
)


def kernel(user_index, item_index, user_emb, item_emb, ub, ib):
    uidx2d = user_index.astype(jnp.int32).reshape(B // CHUNK, CHUNK)
    iidx2d = item_index.astype(jnp.int32).reshape(B // CHUNK, CHUNK)
    uln, iln = _pad_call(user_emb, item_emb)
    ubf = ub.reshape(-1)
    ibf = ib.reshape(-1)
    partials, bias = _make_sc_call()(uidx2d, iidx2d, uln, iln, ubf, ibf)
    out2d = _tc_call(bias.reshape(128, 128), partials)
    return out2d.reshape(B)


# speedup vs baseline: 1.0014x; 1.0014x over previous
import functools

import jax
import jax.numpy as jnp
from jax import lax
from jax.experimental import pallas as pl
from jax.experimental.pallas import tpu as pltpu
from jax.experimental.pallas import tpu_sc as plsc

B = 16384
D = 32
NC = 2
NS = 16
NW = NC * NS
BPW = B // NW
CHUNK = 128
NCHUNK = BPW // CHUNK


def _sc_body(uidx_hbm, iidx_hbm, uln_hbm, iln_hbm, ubf_hbm, ibf_hbm,
             partials_hbm, bias_hbm,
             uidx_v, iidx_v, ugrp_v, igrp_v,
             ubv_v, ibv_v, acc_v, outb_v,
             sem_u, sem_i, sem_ub, sem_ib):
    wid = lax.axis_index("s") * NC + lax.axis_index("c")
    base = wid * BPW
    row0 = wid * NCHUNK

    pltpu.sync_copy(uidx_hbm.at[pl.ds(row0, NCHUNK)], uidx_v)
    pltpu.sync_copy(iidx_hbm.at[pl.ds(row0, NCHUNK)], iidx_v)

    bias_copies = []
    for j in range(NCHUNK):
        bias_copies.append(pltpu.async_copy(
            ubf_hbm.at[uidx_v.at[j]], ubv_v.at[pl.ds(j * CHUNK, CHUNK)],
            sem_ub))
        bias_copies.append(pltpu.async_copy(
            ibf_hbm.at[iidx_v.at[j]], ibv_v.at[pl.ds(j * CHUNK, CHUNK)],
            sem_ib))

    acc = jnp.zeros((16,), jnp.float32)

    for j in range(NCHUNK):
        cu = pltpu.async_copy(uln_hbm.at[uidx_v.at[j]], ugrp_v, sem_u)
        ci = pltpu.async_copy(iln_hbm.at[iidx_v.at[j]], igrp_v, sem_i)
        cu.wait()
        ci.wait()

        def dot_body(n, a):
            u0 = ugrp_v[n, pl.ds(0, 16)]
            v0 = igrp_v[n, pl.ds(0, 16)]
            u1 = ugrp_v[n, pl.ds(16, 16)]
            v1 = igrp_v[n, pl.ds(16, 16)]
            return a + u0 * v0 + u1 * v1

        acc = lax.fori_loop(0, CHUNK, dot_body, acc, unroll=4)

    acc_v[...] = acc
    pltpu.sync_copy(acc_v, partials_hbm.at[wid])

    for c in bias_copies:
        c.wait()
    for j in range(BPW // 16):
        outb_v[pl.ds(j * 16, 16)] = (
            ubv_v[pl.ds(j * 16, 16)] + ibv_v[pl.ds(j * 16, 16)])
    pltpu.sync_copy(outb_v, bias_hbm.at[pl.ds(base, BPW)])


@functools.cache
def _make_sc_call():
    return pl.kernel(
        _sc_body,
        out_type=[
            jax.ShapeDtypeStruct((NW, 16), jnp.float32),
            jax.ShapeDtypeStruct((B,), jnp.float32),
        ],
        mesh=plsc.VectorSubcoreMesh(
            core_axis_name="c", subcore_axis_name="s"),
        compiler_params=pltpu.CompilerParams(
            use_tc_tiling_on_sc=True, needs_layout_passes=False),
        scratch_types=[
            pltpu.VMEM((NCHUNK, CHUNK), jnp.int32),
            pltpu.VMEM((NCHUNK, CHUNK), jnp.int32),
            pltpu.VMEM((CHUNK, 128), jnp.float32),
            pltpu.VMEM((CHUNK, 128), jnp.float32),
            pltpu.VMEM((BPW,), jnp.float32),
            pltpu.VMEM((BPW,), jnp.float32),
            pltpu.VMEM((16,), jnp.float32),
            pltpu.VMEM((BPW,), jnp.float32),
            pltpu.SemaphoreType.DMA,
            pltpu.SemaphoreType.DMA,
            pltpu.SemaphoreType.DMA,
            pltpu.SemaphoreType.DMA,
        ],
    )


PAD_BLK = 8000


def _pad_body(u_in, i_in, u_out, i_out):
    u_out[...] = jnp.pad(u_in[...], ((0, 0), (0, 128 - D)))
    i_out[...] = jnp.pad(i_in[...], ((0, 0), (0, 128 - D)))


_pad_call = pl.pallas_call(
    _pad_body,
    grid=(1000000 // PAD_BLK,),
    in_specs=[
        pl.BlockSpec((PAD_BLK, D), lambda i: (i, 0)),
        pl.BlockSpec((PAD_BLK, D), lambda i: (i, 0)),
    ],
    out_specs=[
        pl.BlockSpec((PAD_BLK, 128), lambda i: (i, 0)),
        pl.BlockSpec((PAD_BLK, 128), lambda i: (i, 0)),
    ],
    out_shape=[
        jax.ShapeDtypeStruct((1000000, 128), jnp.float32),
        jax.ShapeDtypeStruct((1000000, 128), jnp.float32),
    ],
)


def _tc_body(bias_ref, partials_ref, out_ref):
    total = jnp.sum(partials_ref[...])
    out_ref[...] = bias_ref[...] + total


_tc_call = pl.pallas_call(
    _tc_body,
    out_shape=jax.ShapeDtypeStruct((128, 128), jnp.float32),
)


def kernel(user_index, item_index, user_emb, item_emb, ub, ib):
    uidx2d = user_index.astype(jnp.int32).reshape(B // CHUNK, CHUNK)
    iidx2d = item_index.astype(jnp.int32).reshape(B // CHUNK, CHUNK)
    uln, iln = _pad_call(user_emb, item_emb)
    ubf = ub.reshape(-1)
    ibf = ib.reshape(-1)
    partials, bias = _make_sc_call()(uidx2d, iidx2d, uln, iln, ubf, ibf)
    out2d = _tc_call(bias.reshape(128, 128), partials)
    return out2d.reshape(B)


# speedup vs baseline: 1.4630x; 1.4609x over previous
import functools

import jax
import jax.numpy as jnp
from jax import lax
from jax.experimental import pallas as pl
from jax.experimental.pallas import tpu as pltpu
from jax.experimental.pallas import tpu_sc as plsc

B = 16384
D = 32
NC = 2
NS = 16
NW = NC * NS
BPW = B // NW
CHUNK = 128
NCHUNK = BPW // CHUNK


def _sc_body(uidx_hbm, iidx_hbm, uemb_hbm, iemb_hbm, ubf_hbm, ibf_hbm,
             partials_hbm, bias_hbm,
             uidx_v, iidx_v, urows_v, irows_v, ubv_v, ibv_v, acc_v, outb_v,
             sem_u, sem_i, sem_ub, sem_ib):
    wid = lax.axis_index("s") * NC + lax.axis_index("c")
    base = wid * BPW
    row0 = wid * NCHUNK

    pltpu.sync_copy(uidx_hbm.at[pl.ds(row0, NCHUNK)], uidx_v)
    pltpu.sync_copy(iidx_hbm.at[pl.ds(row0, NCHUNK)], iidx_v)

    copies = []
    for j in range(NCHUNK):
        copies.append(pltpu.async_copy(
            uemb_hbm.at[uidx_v.at[j]], urows_v.at[pl.ds(j * CHUNK, CHUNK)],
            sem_u))
        copies.append(pltpu.async_copy(
            iemb_hbm.at[iidx_v.at[j]], irows_v.at[pl.ds(j * CHUNK, CHUNK)],
            sem_i))
        copies.append(pltpu.async_copy(
            ubf_hbm.at[uidx_v.at[j]], ubv_v.at[pl.ds(j * CHUNK, CHUNK)],
            sem_ub))
        copies.append(pltpu.async_copy(
            ibf_hbm.at[iidx_v.at[j]], ibv_v.at[pl.ds(j * CHUNK, CHUNK)],
            sem_ib))
    for c in copies:
        c.wait()

    def dot_body(i, acc):
        u0 = urows_v[i, pl.ds(0, 16)]
        v0 = irows_v[i, pl.ds(0, 16)]
        u1 = urows_v[i, pl.ds(16, 16)]
        v1 = irows_v[i, pl.ds(16, 16)]
        return acc + u0 * v0 + u1 * v1

    acc = lax.fori_loop(0, BPW, dot_body, jnp.zeros((16,), jnp.float32),
                        unroll=4)
    acc_v[...] = acc
    pltpu.sync_copy(acc_v, partials_hbm.at[wid])

    for j in range(BPW // 16):
        outb_v[pl.ds(j * 16, 16)] = (
            ubv_v[pl.ds(j * 16, 16)] + ibv_v[pl.ds(j * 16, 16)])
    pltpu.sync_copy(outb_v, bias_hbm.at[pl.ds(base, BPW)])


@functools.cache
def _make_sc_call():
    return pl.kernel(
        _sc_body,
        out_type=[
            jax.ShapeDtypeStruct((NW, 16), jnp.float32),
            jax.ShapeDtypeStruct((B,), jnp.float32),
        ],
        mesh=plsc.VectorSubcoreMesh(
            core_axis_name="c", subcore_axis_name="s"),
        compiler_params=pltpu.CompilerParams(use_tc_tiling_on_sc=False),
        scratch_types=[
            pltpu.VMEM((NCHUNK, CHUNK), jnp.int32),
            pltpu.VMEM((NCHUNK, CHUNK), jnp.int32),
            pltpu.VMEM((BPW, D), jnp.float32),
            pltpu.VMEM((BPW, D), jnp.float32),
            pltpu.VMEM((BPW,), jnp.float32),
            pltpu.VMEM((BPW,), jnp.float32),
            pltpu.VMEM((16,), jnp.float32),
            pltpu.VMEM((BPW,), jnp.float32),
            pltpu.SemaphoreType.DMA,
            pltpu.SemaphoreType.DMA,
            pltpu.SemaphoreType.DMA,
            pltpu.SemaphoreType.DMA,
        ],
    )


def _tc_body(bias_ref, partials_ref, out_ref):
    total = jnp.sum(partials_ref[...])
    out_ref[...] = bias_ref[...] + total


_tc_call = pl.pallas_call(
    _tc_body,
    out_shape=jax.ShapeDtypeStruct((128, 128), jnp.float32),
)


def kernel(user_index, item_index, user_emb, item_emb, ub, ib):
    uidx2d = user_index.astype(jnp.int32).reshape(B // CHUNK, CHUNK)
    iidx2d = item_index.astype(jnp.int32).reshape(B // CHUNK, CHUNK)
    ubf = ub.reshape(-1)
    ibf = ib.reshape(-1)
    partials, bias = _make_sc_call()(uidx2d, iidx2d, user_emb, item_emb,
                                     ubf, ibf)
    out2d = _tc_call(bias.reshape(128, 128), partials)
    return out2d.reshape(B)
